# baseline (speedup 1.0000x reference)
import functools

import jax
import jax.numpy as jnp
from jax import lax
from jax.experimental import pallas as pl
from jax.experimental.pallas import tpu as pltpu
from jax.experimental.pallas import tpu_sc as plsc

_NC = 2
_NS = 16
_NW = _NC * _NS
_K = 80
_NBUF = 3


def _sc_segment_sum(N, E, D):
    ept = E // _NW
    nchunk = ept // _K
    rpt = N // _NS
    nzero = rpt // _K
    rzero = rpt - nzero * _K
    wchunk = (N // _NS) & ~7
    dtail = N - _NS * wchunk
    assert nchunk >= _NBUF

    mesh = plsc.VectorSubcoreMesh(core_axis_name="c", subcore_axis_name="s")

    @functools.partial(
        pl.kernel,
        mesh=mesh,
        compiler_params=pltpu.CompilerParams(use_tc_tiling_on_sc=True),
        out_type=[
            jax.ShapeDtypeStruct((_NC, N, D), jnp.float32),
            jax.ShapeDtypeStruct((_NC, N), jnp.float32),
        ],
        scratch_types=[
            pltpu.VMEM((nchunk, _K), jnp.int32),
            [pltpu.VMEM((_K,), jnp.int32)] * _NBUF,
            [pltpu.VMEM((_K,), jnp.int32)] * _NBUF,
            [pltpu.VMEM((_K, D), jnp.float32)] * _NBUF,
            pltpu.VMEM((_K,), jnp.float32),
            pltpu.VMEM((wchunk + 16,), jnp.float32),
            [pltpu.SemaphoreType.DMA] * _NBUF,
            [pltpu.SemaphoreType.DMA] * _NBUF,
            pltpu.SemaphoreType.DMA,
            pltpu.VMEM_SHARED((N, D), jnp.float32),
            pltpu.VMEM_SHARED((N,), jnp.float32),
        ],
    )
    def sc(hn_hbm, sd_hbm, out_hbm, deg_hbm,
           sd_v, src_r, dst_r, bufs, ones_v, zdeg_v,
           gsems, ssems, zsem, acc_sh, deg_sh):
        c = lax.axis_index("c")
        s = lax.axis_index("s")
        wid = c * _NS + s

        zero16 = jnp.zeros((16,), jnp.float32)
        mask16 = jnp.full((16,), 0xFFFF, jnp.int32)
        sh16 = jnp.full((16,), 16, jnp.int32)

        pltpu.async_copy(sd_hbm.at[wid], sd_v, gsems[0])

        def zrow(i, carry):
            for j in range(D // 16):
                bufs[0][i, pl.ds(j * 16, 16)] = zero16
            return carry
        lax.fori_loop(0, _K, zrow, 0)

        for i in range(_K // 16):
            ones_v[pl.ds(i * 16, 16)] = jnp.ones((16,), jnp.float32)

        def zdeg(i, carry):
            zdeg_v[pl.ds(i * 16, 16)] = zero16
            return carry
        lax.fori_loop(0, (wchunk + 16) // 16, zdeg, 0)

        base_r = s * wchunk
        nz2 = wchunk // _K
        rz2 = wchunk - nz2 * _K
        zdescs = []
        for q in range(nz2):
            zdescs.append((bufs[0], acc_sh.at[pl.ds(base_r + q * _K, _K)]))
        if rz2:
            zdescs.append((bufs[0].at[pl.ds(0, rz2)],
                           acc_sh.at[pl.ds(base_r + nz2 * _K, rz2)]))
        for sdsc, ddsc in zdescs:
            pltpu.async_copy(sdsc, ddsc, zsem)

        @pl.when(s == _NS - 1)
        def _():
            pltpu.sync_copy(bufs[0].at[pl.ds(0, N - _NS * wchunk)],
                            acc_sh.at[pl.ds(_NS * wchunk, N - _NS * wchunk)])

        wbase = s * wchunk
        pltpu.sync_copy(zdeg_v.at[pl.ds(0, wchunk)],
                        deg_sh.at[pl.ds(wbase, wchunk)])

        @pl.when(s == _NS - 1)
        def _():
            pltpu.sync_copy(zdeg_v.at[pl.ds(0, dtail)],
                            deg_sh.at[pl.ds(_NS * wchunk, dtail)])

        pltpu.make_async_copy(sd_hbm.at[wid], sd_v, gsems[0]).wait()

        def unpack(j, slot):
            for q in range(_K // 16):
                v = sd_v[j, pl.ds(q * 16, 16)]
                src_r[slot][pl.ds(q * 16, 16)] = lax.bitwise_and(v, mask16)
                dst_r[slot][pl.ds(q * 16, 16)] = \
                    lax.shift_right_logical(v, sh16)

        for sdsc, ddsc in zdescs:
            pltpu.make_async_copy(sdsc, ddsc, zsem).wait()

        def gstart(slot):
            pltpu.async_copy(hn_hbm.at[src_r[slot]], bufs[slot], gsems[slot])

        def gwait(slot):
            pltpu.make_async_copy(hn_hbm.at[src_r[slot]], bufs[slot],
                                  gsems[slot]).wait()

        def sstart(slot):
            pltpu.async_copy(bufs[slot], acc_sh.at[dst_r[slot]],
                             ssems[slot], add=True)
            pltpu.async_copy(ones_v, deg_sh.at[dst_r[slot]], ssems[slot],
                             add=True)

        def swait(slot):
            pltpu.make_async_copy(bufs[slot], acc_sh.at[dst_r[slot]],
                                  ssems[slot]).wait()
            pltpu.make_async_copy(ones_v, deg_sh.at[dst_r[slot]],
                                  ssems[slot]).wait()

        for slot in range(_NBUF):
            unpack(slot, slot)
            gstart(slot)

        plsc.subcore_barrier()

        ngroup = nchunk // _NBUF
        nrest = nchunk - ngroup * _NBUF

        def body(t, carry):
            j0 = _NBUF * t
            for slot in range(_NBUF):
                gwait(slot)
                sstart(slot)
            for slot in range(_NBUF):
                swait(slot)
                nj = j0 + _NBUF + slot

                @pl.when(nj < nchunk)
                def _():
                    unpack(nj, slot)
                    gstart(slot)
            return carry
        lax.fori_loop(0, ngroup, body, 0)

        for r in range(nrest):
            gwait(r)
            sstart(r)
        for r in range(nrest):
            swait(r)

        plsc.subcore_barrier()

        tail = _NS * wchunk
        pltpu.sync_copy(acc_sh.at[pl.ds(wbase, wchunk)],
                        out_hbm.at[c, pl.ds(wbase, wchunk)])

        @pl.when(s == _NS - 1)
        def _():
            pltpu.sync_copy(acc_sh.at[pl.ds(tail, N - tail)],
                            out_hbm.at[c, pl.ds(tail, N - tail)])

        @pl.when(s == 0)
        def _():
            pltpu.sync_copy(deg_sh, deg_hbm.at[c])

    return sc


def _tc_body(hs_ref, p_ref, d0_ref, d1_ref, ws_ref, wn_ref, o_ref):
    d = d0_ref[...] + d1_ref[...]
    invd = 1.0 / jnp.maximum(d, 1.0)
    neigh = (p_ref[0] + p_ref[1]) * invd
    z = jnp.dot(hs_ref[...], ws_ref[...], preferred_element_type=jnp.float32)
    z = z + jnp.dot(neigh, wn_ref[...], preferred_element_type=jnp.float32)
    z = jnp.maximum(z, 0.0)
    n2 = jnp.sum(z * z, axis=1, keepdims=True)
    inv = jnp.where(n2 > 0.0, lax.rsqrt(n2), 1.0)
    o_ref[...] = z * inv


def kernel(h_neigh, h_self, edge_index, W_self, W_neigh):
    N, D = h_neigh.shape
    E = edge_index.shape[1]
    nchunk = E // (_NW * _K)

    ei = edge_index.astype(jnp.int32)
    packed = (ei[0] | (ei[1] << 16)).reshape(_NW, nchunk, _K)

    parts, degs = _sc_segment_sum(N, E, D)(h_neigh, packed)
    d0 = degs[0].reshape(N, 1)
    d1 = degs[1].reshape(N, 1)

    blk = 2000
    grid = (N // blk,)
    out = pl.pallas_call(
        _tc_body,
        grid=grid,
        in_specs=[
            pl.BlockSpec((blk, D), lambda i: (i, 0)),
            pl.BlockSpec((_NC, blk, D), lambda i: (0, i, 0)),
            pl.BlockSpec((blk, 1), lambda i: (i, 0)),
            pl.BlockSpec((blk, 1), lambda i: (i, 0)),
            pl.BlockSpec((D, D), lambda i: (0, 0)),
            pl.BlockSpec((D, D), lambda i: (0, 0)),
        ],
        out_specs=pl.BlockSpec((blk, D), lambda i: (i, 0)),
        out_shape=jax.ShapeDtypeStruct((N, D), jnp.float32),
    )(h_self, parts, d0, d1, W_self, W_neigh)
    return out


# speedup vs baseline: 1.0288x; 1.0288x over previous
import functools

import jax
import jax.numpy as jnp
from jax import lax
from jax.experimental import pallas as pl
from jax.experimental.pallas import tpu as pltpu
from jax.experimental.pallas import tpu_sc as plsc

_NC = 2
_NS = 16
_NW = _NC * _NS
_K = 80
_NBUF = 3


def _sc_segment_sum(N, E, D):
    ept = E // _NW
    nchunk = ept // _K
    rpt = N // _NS
    nzero = rpt // _K
    rzero = rpt - nzero * _K
    wchunk = (N // _NS) & ~7
    dtail = N - _NS * wchunk
    assert nchunk >= _NBUF

    mesh = plsc.VectorSubcoreMesh(core_axis_name="c", subcore_axis_name="s")

    @functools.partial(
        pl.kernel,
        mesh=mesh,
        compiler_params=pltpu.CompilerParams(use_tc_tiling_on_sc=False),
        out_type=[
            jax.ShapeDtypeStruct((_NC, N, D), jnp.float32),
            jax.ShapeDtypeStruct((_NC, N), jnp.float32),
        ],
        scratch_types=[
            pltpu.VMEM((nchunk, _K), jnp.int32),
            [pltpu.VMEM((_K,), jnp.int32)] * _NBUF,
            [pltpu.VMEM((_K,), jnp.int32)] * _NBUF,
            [pltpu.VMEM((_K, D), jnp.float32)] * _NBUF,
            pltpu.VMEM((_K,), jnp.float32),
            pltpu.VMEM((wchunk + 16,), jnp.float32),
            [pltpu.SemaphoreType.DMA] * _NBUF,
            [pltpu.SemaphoreType.DMA] * _NBUF,
            pltpu.SemaphoreType.DMA,
            pltpu.VMEM_SHARED((N, D), jnp.float32),
            pltpu.VMEM_SHARED((N,), jnp.float32),
        ],
    )
    def sc(hn_hbm, sd_hbm, out_hbm, deg_hbm,
           sd_v, src_r, dst_r, bufs, ones_v, zdeg_v,
           gsems, ssems, zsem, acc_sh, deg_sh):
        c = lax.axis_index("c")
        s = lax.axis_index("s")
        wid = c * _NS + s

        zero16 = jnp.zeros((16,), jnp.float32)
        mask16 = jnp.full((16,), 0xFFFF, jnp.int32)
        sh16 = jnp.full((16,), 16, jnp.int32)

        pltpu.async_copy(sd_hbm.at[wid], sd_v, gsems[0])

        def zrow(i, carry):
            for j in range(D // 16):
                bufs[0][i, pl.ds(j * 16, 16)] = zero16
            return carry
        lax.fori_loop(0, _K, zrow, 0)

        for i in range(_K // 16):
            ones_v[pl.ds(i * 16, 16)] = jnp.ones((16,), jnp.float32)

        def zdeg(i, carry):
            zdeg_v[pl.ds(i * 16, 16)] = zero16
            return carry
        lax.fori_loop(0, (wchunk + 16) // 16, zdeg, 0)

        base_r = s * rpt
        for q in range(nzero):
            pltpu.async_copy(bufs[0], acc_sh.at[pl.ds(base_r + q * _K, _K)],
                             zsem)
        if rzero:
            pltpu.async_copy(bufs[0].at[pl.ds(0, rzero)],
                             acc_sh.at[pl.ds(base_r + nzero * _K, rzero)],
                             zsem)

        wbase = s * wchunk
        pltpu.sync_copy(zdeg_v.at[pl.ds(0, wchunk)],
                        deg_sh.at[pl.ds(wbase, wchunk)])

        @pl.when(s == _NS - 1)
        def _():
            pltpu.sync_copy(zdeg_v.at[pl.ds(0, dtail)],
                            deg_sh.at[pl.ds(_NS * wchunk, dtail)])

        pltpu.make_async_copy(sd_hbm.at[wid], sd_v, gsems[0]).wait()

        def unpack(j, slot):
            for q in range(_K // 16):
                v = sd_v[j, pl.ds(q * 16, 16)]
                src_r[slot][pl.ds(q * 16, 16)] = lax.bitwise_and(v, mask16)
                dst_r[slot][pl.ds(q * 16, 16)] = \
                    lax.shift_right_logical(v, sh16)

        for q in range(nzero):
            pltpu.make_async_copy(bufs[0],
                                  acc_sh.at[pl.ds(base_r + q * _K, _K)],
                                  zsem).wait()
        if rzero:
            pltpu.make_async_copy(bufs[0].at[pl.ds(0, rzero)],
                                  acc_sh.at[pl.ds(base_r + nzero * _K,
                                                  rzero)],
                                  zsem).wait()

        def gstart(slot):
            pltpu.async_copy(hn_hbm.at[src_r[slot]], bufs[slot], gsems[slot])

        def gwait(slot):
            pltpu.make_async_copy(hn_hbm.at[src_r[slot]], bufs[slot],
                                  gsems[slot]).wait()

        def sstart(slot):
            pltpu.async_copy(bufs[slot], acc_sh.at[dst_r[slot]],
                             ssems[slot], add=True)
            pltpu.async_copy(ones_v, deg_sh.at[dst_r[slot]], ssems[slot],
                             add=True)

        def swait(slot):
            pltpu.make_async_copy(bufs[slot], acc_sh.at[dst_r[slot]],
                                  ssems[slot]).wait()
            pltpu.make_async_copy(ones_v, deg_sh.at[dst_r[slot]],
                                  ssems[slot]).wait()

        for slot in range(_NBUF):
            unpack(slot, slot)
            gstart(slot)

        plsc.subcore_barrier()

        ngroup = nchunk // _NBUF
        nrest = nchunk - ngroup * _NBUF

        def body(t, carry):
            j0 = _NBUF * t
            for slot in range(_NBUF):
                gwait(slot)
                sstart(slot)
            for slot in range(_NBUF):
                swait(slot)
                nj = j0 + _NBUF + slot

                @pl.when(nj < nchunk)
                def _():
                    unpack(nj, slot)
                    gstart(slot)
            return carry
        lax.fori_loop(0, ngroup, body, 0)

        for r in range(nrest):
            gwait(r)
            sstart(r)
        for r in range(nrest):
            swait(r)

        plsc.subcore_barrier()

        tail = _NS * wchunk
        pltpu.sync_copy(acc_sh.at[pl.ds(wbase, wchunk)],
                        out_hbm.at[c, pl.ds(wbase, wchunk)])

        @pl.when(s == _NS - 1)
        def _():
            pltpu.sync_copy(acc_sh.at[pl.ds(tail, N - tail)],
                            out_hbm.at[c, pl.ds(tail, N - tail)])

        @pl.when(s == 0)
        def _():
            pltpu.sync_copy(deg_sh, deg_hbm.at[c])

    return sc


def _tc_body(hs_ref, p_ref, d_ref, ws_ref, wn_ref, o_ref):
    d = d_ref[:, 0:1] + d_ref[:, 1:2]
    invd = 1.0 / jnp.maximum(d, 1.0)
    neigh = (p_ref[0] + p_ref[1]) * invd
    z = jnp.dot(hs_ref[...], ws_ref[...], preferred_element_type=jnp.float32)
    z = z + jnp.dot(neigh, wn_ref[...], preferred_element_type=jnp.float32)
    z = jnp.maximum(z, 0.0)
    n2 = jnp.sum(z * z, axis=1, keepdims=True)
    inv = jnp.where(n2 > 0.0, lax.rsqrt(n2), 1.0)
    o_ref[...] = z * inv


def kernel(h_neigh, h_self, edge_index, W_self, W_neigh):
    N, D = h_neigh.shape
    E = edge_index.shape[1]
    nchunk = E // (_NW * _K)

    ei = edge_index.astype(jnp.int32)
    packed = (ei[0] | (ei[1] << 16)).reshape(_NW, nchunk, _K)

    parts, degs = _sc_segment_sum(N, E, D)(h_neigh, packed)
    d01 = degs.T

    blk = 2000
    grid = (N // blk,)
    out = pl.pallas_call(
        _tc_body,
        grid=grid,
        in_specs=[
            pl.BlockSpec((blk, D), lambda i: (i, 0)),
            pl.BlockSpec((_NC, blk, D), lambda i: (0, i, 0)),
            pl.BlockSpec((blk, _NC), lambda i: (i, 0)),
            pl.BlockSpec((D, D), lambda i: (0, 0)),
            pl.BlockSpec((D, D), lambda i: (0, 0)),
        ],
        out_specs=pl.BlockSpec((blk, D), lambda i: (i, 0)),
        out_shape=jax.ShapeDtypeStruct((N, D), jnp.float32),
    )(h_self, parts, d01, W_self, W_neigh)
    return out
